# baseline (speedup 1.0000x reference)
import functools

import jax
import jax.numpy as jnp
from jax.experimental import pallas as pl

ORDER = 15
DIM = ORDER + 1
K = 256
N_ITER = 3
EPS = 1e-05
PERTURB = 1e-05
MIN_DATA = 1

_BLK = 2048


def _estep_kernel(x_ref, cb_ref, xsq_ref, cbsq_ref,
                  idx_ref, cnt_ref, dist_ref, xout_ref):
    i = pl.program_id(0)
    x = x_ref[...]
    cb = cb_ref[...]

    xc = jax.lax.dot_general(
        x, cb, (((1,), (1,)), ((), ())),
        preferred_element_type=jnp.float32)
    d2 = (xsq_ref[...] - 2.0 * xc) + cbsq_ref[...]

    m = jnp.min(d2, axis=1, keepdims=True)
    lanes = jax.lax.broadcasted_iota(jnp.int32, d2.shape, 1)
    idx = jnp.min(jnp.where(d2 == m, lanes, K), axis=1)

    onehot = (idx[:, None] == lanes)
    oh_f = onehot.astype(jnp.float32)
    cnt_blk = jnp.sum(oh_f, axis=0)
    xq = jax.lax.dot_general(
        oh_f, cb, (((1,), (0,)), ((), ())),
        precision=jax.lax.Precision.HIGHEST,
        preferred_element_type=jnp.float32)
    diff = x - xq
    dist_blk = jnp.sum(diff * diff)

    idx_ref[...] = idx[:, None]
    xout_ref[...] = x

    @pl.when(i == 0)
    def _init():
        cnt_ref[...] = jnp.zeros_like(cnt_ref)
        dist_ref[...] = jnp.zeros_like(dist_ref)

    cnt_ref[...] += cnt_blk[None, :]
    dist_ref[...] += dist_blk


@jax.jit
def _estep(x, cb, xsq, cbsq):
    T = x.shape[0]
    grid = T // _BLK
    idx, cnt, dist, xout = pl.pallas_call(
        _estep_kernel,
        grid=(grid,),
        in_specs=[
            pl.BlockSpec((_BLK, DIM), lambda i: (i, 0)),
            pl.BlockSpec((K, DIM), lambda i: (0, 0)),
            pl.BlockSpec((_BLK, 1), lambda i: (i, 0)),
            pl.BlockSpec((1, K), lambda i: (0, 0)),
        ],
        out_specs=[
            pl.BlockSpec((_BLK, 1), lambda i: (i, 0)),
            pl.BlockSpec((1, K), lambda i: (0, 0)),
            pl.BlockSpec((1, 1), lambda i: (0, 0)),
            pl.BlockSpec((_BLK, DIM), lambda i: (i, 0)),
        ],
        out_shape=[
            jax.ShapeDtypeStruct((T, 1), jnp.int32),
            jax.ShapeDtypeStruct((1, K), jnp.float32),
            jax.ShapeDtypeStruct((1, 1), jnp.float32),
            jax.ShapeDtypeStruct((T, DIM), jnp.float32),
        ],
    )(x, cb, xsq, cbsq)
    return idx[:, 0], cnt[0], dist[0, 0], xout


def kernel(x):
    T, dim = x.shape
    mean = jnp.sum(x, axis=0) / T
    codebook = jnp.full((K, dim), 1e10, dtype=x.dtype).at[0].set(mean)
    rkey = jax.random.key(42)
    distance = jnp.asarray(jnp.inf, dtype=x.dtype)
    xsq = jnp.sum(x * x, axis=1, keepdims=True)
    curr, nxt = 1, 2
    while nxt <= K:
        rkey, sub = jax.random.split(rkey)
        r = jax.random.normal(sub, (curr, dim), dtype=x.dtype) * PERTURB
        codebook = codebook.at[curr:nxt].set(codebook[:curr] - r)
        codebook = codebook.at[:curr].add(r)
        curr, nxt = nxt, nxt * 2
        prev_distance = distance
        done = jnp.asarray(False)
        for n in range(N_ITER):
            cb_curr = codebook[:curr]
            cbsq_real = jnp.sum(cb_curr * cb_curr, axis=1)
            if curr < K:
                cbsq = jnp.concatenate(
                    [cbsq_real,
                     jnp.full((K - curr,), 2e21, dtype=x.dtype)])
            else:
                cbsq = cbsq_real
            indices, cnt, dsum, x_rm = _estep(x, codebook, xsq, cbsq[None, :])
            d_new = dsum / T
            distance = jnp.where(done, distance, d_new)
            change = jnp.abs(prev_distance - d_new)
            if n:
                done = jnp.logical_or(done, change / (d_new + 1e-16) < EPS)
            prev_distance = jnp.where(done, prev_distance, d_new)
            n_data = cnt[:curr]
            mask = n_data >= MIN_DATA
            centroids = jax.ops.segment_sum(x_rm, indices, num_segments=curr)
            centroids = jnp.where(
                mask[:, None], centroids / jnp.maximum(n_data, 1.0)[:, None],
                centroids)
            m = jnp.argmax(n_data)
            rkey2, sub2 = jax.random.split(rkey)
            rf = jax.random.normal(sub2, (curr, dim), dtype=x.dtype) * PERTURB
            bad = ~mask
            nbad = jnp.sum(bad).astype(x.dtype)
            centroids2 = jnp.where(bad[:, None], centroids[m] - rf, centroids)
            r_mean = jnp.sum(rf * bad[:, None].astype(x.dtype), axis=0) / \
                jnp.maximum(nbad, 1.0)
            centroids_alt = centroids2.at[m].add(r_mean)
            take_branch = jnp.logical_and(jnp.logical_not(done), jnp.any(bad))
            centroids = jnp.where(take_branch, centroids_alt, centroids)
            rkey = jax.random.wrap_key_data(
                jnp.where(take_branch, jax.random.key_data(rkey2),
                          jax.random.key_data(rkey)))
            codebook = codebook.at[:curr].set(
                jnp.where(done, codebook[:curr], centroids))
    return codebook, jnp.asarray(distance, dtype=x.dtype)


# speedup vs baseline: 1.1434x; 1.1434x over previous
import functools

import jax
import jax.numpy as jnp
from jax.experimental import pallas as pl

ORDER = 15
DIM = ORDER + 1
K = 256
N_ITER = 3
EPS = 1e-05
PERTURB = 1e-05
MIN_DATA = 1

_BLK = 2048


def _estep_kernel(x_ref, cb_ref, xsq_ref, cbsq_ref,
                  idx_ref, cnt_ref, dist_ref, *, width):
    i = pl.program_id(0)
    x = x_ref[...]
    cb = cb_ref[...]

    xc = jax.lax.dot_general(
        x, cb, (((1,), (1,)), ((), ())),
        preferred_element_type=jnp.float32)
    d2 = (xsq_ref[...] - 2.0 * xc) + cbsq_ref[...]

    m = jnp.min(d2, axis=1, keepdims=True)
    lanes = jax.lax.broadcasted_iota(jnp.int32, d2.shape, 1)
    idx = jnp.min(jnp.where(d2 == m, lanes, width), axis=1)

    onehot = (idx[:, None] == lanes)
    oh_f = onehot.astype(jnp.float32)
    cnt_blk = jnp.sum(oh_f, axis=0)
    xq = jax.lax.dot_general(
        oh_f, cb, (((1,), (0,)), ((), ())),
        precision=jax.lax.Precision.HIGHEST,
        preferred_element_type=jnp.float32)
    diff = x - xq
    dist_blk = jnp.sum(diff * diff)

    idx_ref[...] = idx[:, None]

    @pl.when(i == 0)
    def _init():
        cnt_ref[...] = jnp.zeros_like(cnt_ref)
        dist_ref[...] = jnp.zeros_like(dist_ref)

    cnt_ref[...] += cnt_blk[None, :]
    dist_ref[...] += dist_blk


@functools.partial(jax.jit, static_argnames=("width",))
def _estep(x, cb, xsq, cbsq, width):
    T = x.shape[0]
    grid = T // _BLK
    idx, cnt, dist = pl.pallas_call(
        functools.partial(_estep_kernel, width=width),
        grid=(grid,),
        in_specs=[
            pl.BlockSpec((_BLK, DIM), lambda i: (i, 0)),
            pl.BlockSpec((width, DIM), lambda i: (0, 0)),
            pl.BlockSpec((_BLK, 1), lambda i: (i, 0)),
            pl.BlockSpec((1, width), lambda i: (0, 0)),
        ],
        out_specs=[
            pl.BlockSpec((_BLK, 1), lambda i: (i, 0)),
            pl.BlockSpec((1, width), lambda i: (0, 0)),
            pl.BlockSpec((1, 1), lambda i: (0, 0)),
        ],
        out_shape=[
            jax.ShapeDtypeStruct((T, 1), jnp.int32),
            jax.ShapeDtypeStruct((1, width), jnp.float32),
            jax.ShapeDtypeStruct((1, 1), jnp.float32),
        ],
    )(x, cb, xsq, cbsq)
    return idx[:, 0], cnt[0], dist[0, 0]


def kernel(x):
    T, dim = x.shape
    mean = jnp.sum(x, axis=0) / T
    codebook = jnp.full((K, dim), 1e10, dtype=x.dtype).at[0].set(mean)
    rkey = jax.random.key(42)
    distance = jnp.asarray(jnp.inf, dtype=x.dtype)
    xsq = jnp.sum(x * x, axis=1, keepdims=True)
    curr, nxt = 1, 2
    while nxt <= K:
        rkey, sub = jax.random.split(rkey)
        r = jax.random.normal(sub, (curr, dim), dtype=x.dtype) * PERTURB
        codebook = codebook.at[curr:nxt].set(codebook[:curr] - r)
        codebook = codebook.at[:curr].add(r)
        curr, nxt = nxt, nxt * 2
        prev_distance = distance
        done = jnp.asarray(False)
        width = 128 if curr <= 128 else K
        for n in range(N_ITER):
            cb_curr = codebook[:curr]
            cbsq_real = jnp.sum(cb_curr * cb_curr, axis=1)
            if curr < width:
                cbsq = jnp.concatenate(
                    [cbsq_real,
                     jnp.full((width - curr,), 2e21, dtype=x.dtype)])
            else:
                cbsq = cbsq_real
            indices, cnt, dsum = _estep(
                x, codebook[:width], xsq, cbsq[None, :], width)
            d_new = dsum / T
            distance = jnp.where(done, distance, d_new)
            change = jnp.abs(prev_distance - d_new)
            if n:
                done = jnp.logical_or(done, change / (d_new + 1e-16) < EPS)
            prev_distance = jnp.where(done, prev_distance, d_new)
            n_data = cnt[:curr]
            mask = n_data >= MIN_DATA
            centroids = jax.ops.segment_sum(x, indices, num_segments=curr)
            centroids = jnp.where(
                mask[:, None], centroids / jnp.maximum(n_data, 1.0)[:, None],
                centroids)
            m = jnp.argmax(n_data)
            rkey2, sub2 = jax.random.split(rkey)
            rf = jax.random.normal(sub2, (curr, dim), dtype=x.dtype) * PERTURB
            bad = ~mask
            nbad = jnp.sum(bad).astype(x.dtype)
            centroids2 = jnp.where(bad[:, None], centroids[m] - rf, centroids)
            r_mean = jnp.sum(rf * bad[:, None].astype(x.dtype), axis=0) / \
                jnp.maximum(nbad, 1.0)
            centroids_alt = centroids2.at[m].add(r_mean)
            take_branch = jnp.logical_and(jnp.logical_not(done), jnp.any(bad))
            centroids = jnp.where(take_branch, centroids_alt, centroids)
            rkey = jax.random.wrap_key_data(
                jnp.where(take_branch, jax.random.key_data(rkey2),
                          jax.random.key_data(rkey)))
            codebook = codebook.at[:curr].set(
                jnp.where(done, codebook[:curr], centroids))
    return codebook, jnp.asarray(distance, dtype=x.dtype)


# speedup vs baseline: 1.1985x; 1.0482x over previous
import functools

import jax
import jax.numpy as jnp
from jax.experimental import pallas as pl

ORDER = 15
DIM = ORDER + 1
K = 256
N_ITER = 3
EPS = 1e-05
PERTURB = 1e-05
MIN_DATA = 1

_BLK = 4096


def _estep_kernel(x_ref, cb_ref, xsq_ref, cbsq_ref,
                  idx_ref, cnt_ref, dist_ref, *, width):
    i = pl.program_id(0)
    x = x_ref[...]
    cb = cb_ref[...]

    xc = jax.lax.dot_general(
        x, cb, (((1,), (1,)), ((), ())),
        preferred_element_type=jnp.float32)
    d2 = (xsq_ref[...] - 2.0 * xc) + cbsq_ref[...]

    m = jnp.min(d2, axis=1, keepdims=True)
    lanes = jax.lax.broadcasted_iota(jnp.int32, d2.shape, 1)
    idx = jnp.min(jnp.where(d2 == m, lanes, width), axis=1)

    onehot = (idx[:, None] == lanes)
    oh_f = onehot.astype(jnp.float32)
    cnt_blk = jnp.sum(oh_f, axis=0)
    xq = jax.lax.dot_general(
        oh_f, cb, (((1,), (0,)), ((), ())),
        precision=jax.lax.Precision.HIGHEST,
        preferred_element_type=jnp.float32)
    diff = x - xq
    dist_blk = jnp.sum(diff * diff)

    idx_ref[...] = idx[:, None]

    @pl.when(i == 0)
    def _init():
        cnt_ref[...] = jnp.zeros_like(cnt_ref)
        dist_ref[...] = jnp.zeros_like(dist_ref)

    cnt_ref[...] += cnt_blk[None, :]
    dist_ref[...] += dist_blk


@functools.partial(jax.jit, static_argnames=("width",))
def _estep(x, cb, xsq, cbsq, width):
    T = x.shape[0]
    grid = T // _BLK
    idx, cnt, dist = pl.pallas_call(
        functools.partial(_estep_kernel, width=width),
        grid=(grid,),
        in_specs=[
            pl.BlockSpec((_BLK, DIM), lambda i: (i, 0)),
            pl.BlockSpec((width, DIM), lambda i: (0, 0)),
            pl.BlockSpec((_BLK, 1), lambda i: (i, 0)),
            pl.BlockSpec((1, width), lambda i: (0, 0)),
        ],
        out_specs=[
            pl.BlockSpec((_BLK, 1), lambda i: (i, 0)),
            pl.BlockSpec((1, width), lambda i: (0, 0)),
            pl.BlockSpec((1, 1), lambda i: (0, 0)),
        ],
        out_shape=[
            jax.ShapeDtypeStruct((T, 1), jnp.int32),
            jax.ShapeDtypeStruct((1, width), jnp.float32),
            jax.ShapeDtypeStruct((1, 1), jnp.float32),
        ],
    )(x, cb, xsq, cbsq)
    return idx[:, 0], cnt[0], dist[0, 0]


def kernel(x):
    T, dim = x.shape
    mean = jnp.sum(x, axis=0) / T
    codebook = jnp.full((K, dim), 1e10, dtype=x.dtype).at[0].set(mean)
    rkey = jax.random.key(42)
    distance = jnp.asarray(jnp.inf, dtype=x.dtype)
    xsq = jnp.sum(x * x, axis=1, keepdims=True)
    curr, nxt = 1, 2
    while nxt <= K:
        rkey, sub = jax.random.split(rkey)
        r = jax.random.normal(sub, (curr, dim), dtype=x.dtype) * PERTURB
        codebook = codebook.at[curr:nxt].set(codebook[:curr] - r)
        codebook = codebook.at[:curr].add(r)
        curr, nxt = nxt, nxt * 2
        prev_distance = distance
        done = jnp.asarray(False)
        width = 128 if curr <= 128 else K
        for n in range(N_ITER):
            cb_curr = codebook[:curr]
            cbsq_real = jnp.sum(cb_curr * cb_curr, axis=1)
            if curr < width:
                cbsq = jnp.concatenate(
                    [cbsq_real,
                     jnp.full((width - curr,), 2e21, dtype=x.dtype)])
            else:
                cbsq = cbsq_real
            indices, cnt, dsum = _estep(
                x, codebook[:width], xsq, cbsq[None, :], width)
            d_new = dsum / T
            distance = jnp.where(done, distance, d_new)
            change = jnp.abs(prev_distance - d_new)
            if n:
                done = jnp.logical_or(done, change / (d_new + 1e-16) < EPS)
            prev_distance = jnp.where(done, prev_distance, d_new)
            n_data = cnt[:curr]
            mask = n_data >= MIN_DATA
            centroids = jax.ops.segment_sum(x, indices, num_segments=curr)
            centroids = jnp.where(
                mask[:, None], centroids / jnp.maximum(n_data, 1.0)[:, None],
                centroids)
            m = jnp.argmax(n_data)
            rkey2, sub2 = jax.random.split(rkey)
            rf = jax.random.normal(sub2, (curr, dim), dtype=x.dtype) * PERTURB
            bad = ~mask
            nbad = jnp.sum(bad).astype(x.dtype)
            centroids2 = jnp.where(bad[:, None], centroids[m] - rf, centroids)
            r_mean = jnp.sum(rf * bad[:, None].astype(x.dtype), axis=0) / \
                jnp.maximum(nbad, 1.0)
            centroids_alt = centroids2.at[m].add(r_mean)
            take_branch = jnp.logical_and(jnp.logical_not(done), jnp.any(bad))
            centroids = jnp.where(take_branch, centroids_alt, centroids)
            rkey = jax.random.wrap_key_data(
                jnp.where(take_branch, jax.random.key_data(rkey2),
                          jax.random.key_data(rkey)))
            codebook = codebook.at[:curr].set(
                jnp.where(done, codebook[:curr], centroids))
    return codebook, jnp.asarray(distance, dtype=x.dtype)


# speedup vs baseline: 1.2103x; 1.0099x over previous
import functools

import jax
import jax.numpy as jnp
from jax.experimental import pallas as pl

ORDER = 15
DIM = ORDER + 1
K = 256
N_ITER = 3
EPS = 1e-05
PERTURB = 1e-05
MIN_DATA = 1

_BLK = 8192


def _estep_kernel(x_ref, cb_ref, xsq_ref, cbsq_ref,
                  idx_ref, cnt_ref, dist_ref, *, width):
    i = pl.program_id(0)
    x = x_ref[...]
    cb = cb_ref[...]

    xc = jax.lax.dot_general(
        x, cb, (((1,), (1,)), ((), ())),
        preferred_element_type=jnp.float32)
    d2 = (xsq_ref[...] - 2.0 * xc) + cbsq_ref[...]

    m = jnp.min(d2, axis=1, keepdims=True)
    lanes = jax.lax.broadcasted_iota(jnp.int32, d2.shape, 1)
    idx = jnp.min(jnp.where(d2 == m, lanes, width), axis=1)

    onehot = (idx[:, None] == lanes)
    oh_f = onehot.astype(jnp.float32)
    cnt_blk = jnp.sum(oh_f, axis=0)
    xq = jax.lax.dot_general(
        oh_f, cb, (((1,), (0,)), ((), ())),
        precision=jax.lax.Precision.HIGHEST,
        preferred_element_type=jnp.float32)
    diff = x - xq
    dist_blk = jnp.sum(diff * diff)

    idx_ref[...] = idx[:, None]

    @pl.when(i == 0)
    def _init():
        cnt_ref[...] = jnp.zeros_like(cnt_ref)
        dist_ref[...] = jnp.zeros_like(dist_ref)

    cnt_ref[...] += cnt_blk[None, :]
    dist_ref[...] += dist_blk


@functools.partial(jax.jit, static_argnames=("width",))
def _estep(x, cb, xsq, cbsq, width):
    T = x.shape[0]
    grid = T // _BLK
    idx, cnt, dist = pl.pallas_call(
        functools.partial(_estep_kernel, width=width),
        grid=(grid,),
        in_specs=[
            pl.BlockSpec((_BLK, DIM), lambda i: (i, 0)),
            pl.BlockSpec((width, DIM), lambda i: (0, 0)),
            pl.BlockSpec((_BLK, 1), lambda i: (i, 0)),
            pl.BlockSpec((1, width), lambda i: (0, 0)),
        ],
        out_specs=[
            pl.BlockSpec((_BLK, 1), lambda i: (i, 0)),
            pl.BlockSpec((1, width), lambda i: (0, 0)),
            pl.BlockSpec((1, 1), lambda i: (0, 0)),
        ],
        out_shape=[
            jax.ShapeDtypeStruct((T, 1), jnp.int32),
            jax.ShapeDtypeStruct((1, width), jnp.float32),
            jax.ShapeDtypeStruct((1, 1), jnp.float32),
        ],
    )(x, cb, xsq, cbsq)
    return idx[:, 0], cnt[0], dist[0, 0]


def kernel(x):
    T, dim = x.shape
    mean = jnp.sum(x, axis=0) / T
    codebook = jnp.full((K, dim), 1e10, dtype=x.dtype).at[0].set(mean)
    rkey = jax.random.key(42)
    distance = jnp.asarray(jnp.inf, dtype=x.dtype)
    xsq = jnp.sum(x * x, axis=1, keepdims=True)
    curr, nxt = 1, 2
    while nxt <= K:
        rkey, sub = jax.random.split(rkey)
        r = jax.random.normal(sub, (curr, dim), dtype=x.dtype) * PERTURB
        codebook = codebook.at[curr:nxt].set(codebook[:curr] - r)
        codebook = codebook.at[:curr].add(r)
        curr, nxt = nxt, nxt * 2
        prev_distance = distance
        done = jnp.asarray(False)
        width = 128 if curr <= 128 else K
        for n in range(N_ITER):
            cb_curr = codebook[:curr]
            cbsq_real = jnp.sum(cb_curr * cb_curr, axis=1)
            if curr < width:
                cbsq = jnp.concatenate(
                    [cbsq_real,
                     jnp.full((width - curr,), 2e21, dtype=x.dtype)])
            else:
                cbsq = cbsq_real
            indices, cnt, dsum = _estep(
                x, codebook[:width], xsq, cbsq[None, :], width)
            d_new = dsum / T
            distance = jnp.where(done, distance, d_new)
            change = jnp.abs(prev_distance - d_new)
            if n:
                done = jnp.logical_or(done, change / (d_new + 1e-16) < EPS)
            prev_distance = jnp.where(done, prev_distance, d_new)
            n_data = cnt[:curr]
            mask = n_data >= MIN_DATA
            centroids = jax.ops.segment_sum(x, indices, num_segments=curr)
            centroids = jnp.where(
                mask[:, None], centroids / jnp.maximum(n_data, 1.0)[:, None],
                centroids)
            m = jnp.argmax(n_data)
            rkey2, sub2 = jax.random.split(rkey)
            rf = jax.random.normal(sub2, (curr, dim), dtype=x.dtype) * PERTURB
            bad = ~mask
            nbad = jnp.sum(bad).astype(x.dtype)
            centroids2 = jnp.where(bad[:, None], centroids[m] - rf, centroids)
            r_mean = jnp.sum(rf * bad[:, None].astype(x.dtype), axis=0) / \
                jnp.maximum(nbad, 1.0)
            centroids_alt = centroids2.at[m].add(r_mean)
            take_branch = jnp.logical_and(jnp.logical_not(done), jnp.any(bad))
            centroids = jnp.where(take_branch, centroids_alt, centroids)
            rkey = jax.random.wrap_key_data(
                jnp.where(take_branch, jax.random.key_data(rkey2),
                          jax.random.key_data(rkey)))
            codebook = codebook.at[:curr].set(
                jnp.where(done, codebook[:curr], centroids))
    return codebook, jnp.asarray(distance, dtype=x.dtype)
